# baseline (speedup 1.0000x reference)
import functools

import jax
import jax.numpy as jnp
from jax import lax
from jax.experimental import pallas as pl
from jax.experimental.pallas import tpu as pltpu
from jax.experimental.pallas import tpu_sc as plsc

PDIM = 128
CHUNK = 128
NBUF = 4


@functools.lru_cache(maxsize=None)
def _build(n_total, n_workers, n_chunks):
    per_worker = n_chunks * CHUNK
    mesh = plsc.VectorSubcoreMesh(core_axis_name="c", subcore_axis_name="s")

    @functools.partial(
        pl.kernel,
        mesh=mesh,
        out_type=jax.ShapeDtypeStruct((n_total, PDIM), jnp.float32),
        scratch_types=[
            pltpu.VMEM((n_chunks, CHUNK), jnp.int32),
        ]
        + [pltpu.VMEM((CHUNK, PDIM), jnp.float32)] * NBUF
        + [pltpu.SemaphoreType.DMA] * (2 * NBUF),
    )
    def k(idx_hbm, table_hbm, out_hbm, idx_v, *rest):
        bufs = rest[:NBUF]
        gsems = rest[NBUF : 2 * NBUF]
        wsems = rest[2 * NBUF :]
        wid = lax.axis_index("s") * 2 + lax.axis_index("c")
        pltpu.sync_copy(idx_hbm.at[wid], idx_v)
        base = wid * per_worker

        def fire_gather(t, r):
            pltpu.async_copy(table_hbm.at[idx_v.at[t]], bufs[r], gsems[r])

        def drain_gather(r):
            pltpu.make_async_copy(
                table_hbm.at[idx_v.at[0]], bufs[r], gsems[r]
            ).wait()

        def fire_write(t, r):
            pltpu.async_copy(
                bufs[r], out_hbm.at[pl.ds(base + t * CHUNK, CHUNK)], wsems[r]
            )

        def drain_write(r):
            pltpu.make_async_copy(
                bufs[r], out_hbm.at[pl.ds(0, CHUNK)], wsems[r]
            ).wait()

        fire_gather(0, 0)
        fire_gather(1, 1)

        def step(t, r):
            @pl.when(t >= 2)
            def _():
                drain_write((r - 2) % NBUF)

            @pl.when(t + 2 < n_chunks)
            def _():
                fire_gather(t + 2, (r + 2) % NBUF)

            drain_gather(r)
            fire_write(t, r)

        def quad(i, carry):
            for r in range(NBUF):
                step(NBUF * i + r, r)
            return carry

        lax.fori_loop(0, n_chunks // NBUF, quad, 0)
        drain_write(2)
        drain_write(3)

    return k


def kernel(token_ids, weights):
    b, t = token_ids.shape
    dim = weights.shape[1]
    n_total = b * t
    n_workers = 32
    n_chunks = n_total // (n_workers * CHUNK)
    assert n_total % (n_workers * CHUNK) == 0 and n_chunks % NBUF == 0
    idx = token_ids.astype(jnp.int32).reshape(n_workers, n_chunks, CHUNK)
    nv = weights.shape[0]
    cuts = [0, 250112, 500224, 750336, nv]
    wpad = jnp.concatenate(
        [
            jnp.pad(weights[cuts[i] : cuts[i + 1]], ((0, 0), (0, PDIM - dim)))
            for i in range(4)
        ],
        axis=0,
    )
    out = _build(n_total, n_workers, n_chunks)(idx, wpad)
    return out[:, :dim].reshape(b, t, dim)


# speedup vs baseline: 1.6828x; 1.6828x over previous
import functools

import jax
import jax.numpy as jnp
from jax import lax
from jax.experimental import pallas as pl
from jax.experimental.pallas import tpu as pltpu
from jax.experimental.pallas import tpu_sc as plsc

PDIM = 128
CHUNK = 128
NBUF = 4


@functools.lru_cache(maxsize=None)
def _build(n_total, n_workers, n_chunks):
    per_worker = n_chunks * CHUNK
    mesh = plsc.VectorSubcoreMesh(core_axis_name="c", subcore_axis_name="s")

    @functools.partial(
        pl.kernel,
        mesh=mesh,
        out_type=jax.ShapeDtypeStruct((n_total, PDIM), jnp.float32),
        scratch_types=[
            pltpu.VMEM((n_chunks, CHUNK), jnp.int32),
        ]
        + [pltpu.VMEM((CHUNK, PDIM), jnp.float32)] * NBUF
        + [pltpu.SemaphoreType.DMA] * (2 * NBUF),
    )
    def k(idx_hbm, table_hbm, out_hbm, idx_v, *rest):
        bufs = rest[:NBUF]
        gsems = rest[NBUF : 2 * NBUF]
        wsems = rest[2 * NBUF :]
        wid = lax.axis_index("s") * 2 + lax.axis_index("c")
        pltpu.sync_copy(idx_hbm.at[wid], idx_v)
        base = wid * per_worker

        def fire_gather(t, r):
            pltpu.async_copy(table_hbm.at[idx_v.at[t]], bufs[r], gsems[r])

        def drain_gather(r):
            pltpu.make_async_copy(
                table_hbm.at[idx_v.at[0]], bufs[r], gsems[r]
            ).wait()

        def fire_write(t, r):
            pltpu.async_copy(
                bufs[r], out_hbm.at[pl.ds(base + t * CHUNK, CHUNK)], wsems[r]
            )

        def drain_write(r):
            pltpu.make_async_copy(
                bufs[r], out_hbm.at[pl.ds(0, CHUNK)], wsems[r]
            ).wait()

        fire_gather(0, 0)
        fire_gather(1, 1)

        def step(t, r):
            @pl.when(t >= 2)
            def _():
                drain_write((r - 2) % NBUF)

            @pl.when(t + 2 < n_chunks)
            def _():
                fire_gather(t + 2, (r + 2) % NBUF)

            drain_gather(r)
            fire_write(t, r)

        def quad(i, carry):
            for r in range(NBUF):
                step(NBUF * i + r, r)
            return carry

        lax.fori_loop(0, n_chunks // NBUF, quad, 0)
        drain_write(2)
        drain_write(3)

    return k


def kernel(token_ids, weights):
    b, t = token_ids.shape
    dim = weights.shape[1]
    n_total = b * t
    n_workers = 32
    n_chunks = n_total // (n_workers * CHUNK)
    assert n_total % (n_workers * CHUNK) == 0 and n_chunks % NBUF == 0
    idx = token_ids.astype(jnp.int32).reshape(n_workers, n_chunks, CHUNK)
    wpad = jnp.pad(weights, ((0, 0), (0, PDIM - dim)))
    out = _build(n_total, n_workers, n_chunks)(idx, wpad)
    return out[:, :dim].reshape(b, t, dim)


# speedup vs baseline: 1.6845x; 1.0010x over previous
import functools

import jax
import jax.numpy as jnp
from jax import lax
from jax.experimental import pallas as pl
from jax.experimental.pallas import tpu as pltpu
from jax.experimental.pallas import tpu_sc as plsc

PDIM = 128
DIM = 64
CHUNK = 128
NBUF = 4
LANES = 16


@functools.lru_cache(maxsize=None)
def _build(n_total, n_workers, n_chunks):
    per_worker = n_chunks * CHUNK
    mesh = plsc.VectorSubcoreMesh(core_axis_name="c", subcore_axis_name="s")

    @functools.partial(
        pl.kernel,
        mesh=mesh,
        out_type=jax.ShapeDtypeStruct((n_total, DIM), jnp.float32),
        scratch_types=[
            pltpu.VMEM((n_chunks, CHUNK), jnp.int32),
        ]
        + [pltpu.VMEM((CHUNK, PDIM), jnp.float32)] * NBUF
        + [pltpu.VMEM((CHUNK, DIM), jnp.float32)] * 2
        + [pltpu.SemaphoreType.DMA] * (NBUF + 2),
    )
    def k(idx_hbm, table_hbm, out_hbm, idx_v, *rest):
        gbufs = rest[:NBUF]
        cbufs = rest[NBUF : NBUF + 2]
        gsems = rest[NBUF + 2 : 2 * NBUF + 2]
        wsems = rest[2 * NBUF + 2 :]
        wid = lax.axis_index("s") * 2 + lax.axis_index("c")
        pltpu.sync_copy(idx_hbm.at[wid], idx_v)
        base = wid * per_worker

        def fire_gather(t, r):
            pltpu.async_copy(table_hbm.at[idx_v.at[t]], gbufs[r], gsems[r])

        def drain_gather(r):
            pltpu.make_async_copy(
                table_hbm.at[idx_v.at[0]], gbufs[r], gsems[r]
            ).wait()

        def fire_write(t, p):
            pltpu.async_copy(
                cbufs[p], out_hbm.at[pl.ds(base + t * CHUNK, CHUNK)], wsems[p]
            )

        def drain_write(p):
            pltpu.make_async_copy(
                cbufs[p], out_hbm.at[pl.ds(0, CHUNK)], wsems[p]
            ).wait()

        def compact(r, p):
            g, c = gbufs[r], cbufs[p]

            def row(i, carry):
                for u in range(8):
                    tok = i * 8 + u
                    for j in range(DIM // LANES):
                        c[tok, pl.ds(LANES * j, LANES)] = g[
                            tok, pl.ds(LANES * j, LANES)
                        ]
                return carry

            lax.fori_loop(0, CHUNK // 8, row, 0)

        fire_gather(0, 0)
        fire_gather(1, 1)

        def step(t, r):
            p = r % 2
            @pl.when(t >= 2)
            def _():
                drain_write(p)

            @pl.when(t + 2 < n_chunks)
            def _():
                fire_gather(t + 2, (r + 2) % NBUF)

            drain_gather(r)
            compact(r, p)
            fire_write(t, p)

        def quad(i, carry):
            for r in range(NBUF):
                step(NBUF * i + r, r)
            return carry

        lax.fori_loop(0, n_chunks // NBUF, quad, 0)
        drain_write(0)
        drain_write(1)

    return k


def kernel(token_ids, weights):
    b, t = token_ids.shape
    dim = weights.shape[1]
    n_total = b * t
    n_workers = 32
    n_chunks = n_total // (n_workers * CHUNK)
    assert n_total % (n_workers * CHUNK) == 0 and n_chunks % NBUF == 0
    idx = token_ids.astype(jnp.int32).reshape(n_workers, n_chunks, CHUNK)
    wpad = jnp.pad(weights, ((0, 0), (0, PDIM - dim)))
    out = _build(n_total, n_workers, n_chunks)(idx, wpad)
    return out.reshape(b, t, dim)


# speedup vs baseline: 1.6860x; 1.0009x over previous
import functools

import jax
import jax.numpy as jnp
from jax import lax
from jax.experimental import pallas as pl
from jax.experimental.pallas import tpu as pltpu
from jax.experimental.pallas import tpu_sc as plsc

PDIM = 128
CHUNK = 128
NBUF = 4


@functools.lru_cache(maxsize=None)
def _build(n_total, n_workers, n_chunks):
    per_worker = n_chunks * CHUNK
    mesh = plsc.VectorSubcoreMesh(core_axis_name="c", subcore_axis_name="s")

    @functools.partial(
        pl.kernel,
        mesh=mesh,
        out_type=jax.ShapeDtypeStruct((n_total, PDIM), jnp.float32),
        scratch_types=[
            pltpu.VMEM((n_chunks, CHUNK), jnp.int32),
        ]
        + [pltpu.VMEM((CHUNK, PDIM), jnp.float32)] * NBUF
        + [pltpu.SemaphoreType.DMA] * (2 * NBUF),
    )
    def k(idx_hbm, table_hbm, out_hbm, idx_v, *rest):
        bufs = rest[:NBUF]
        gsems = rest[NBUF : 2 * NBUF]
        wsems = rest[2 * NBUF :]
        wid = lax.axis_index("s") * 2 + lax.axis_index("c")
        pltpu.sync_copy(idx_hbm.at[wid], idx_v)
        base = wid * per_worker

        def fire_gather(t, r):
            pltpu.async_copy(table_hbm.at[idx_v.at[t]], bufs[r], gsems[r])

        def drain_gather(r):
            pltpu.make_async_copy(
                table_hbm.at[idx_v.at[0]], bufs[r], gsems[r]
            ).wait()

        def fire_write(t, r):
            pltpu.async_copy(
                bufs[r], out_hbm.at[pl.ds(base + t * CHUNK, CHUNK)], wsems[r]
            )

        def drain_write(r):
            pltpu.make_async_copy(
                bufs[r], out_hbm.at[pl.ds(0, CHUNK)], wsems[r]
            ).wait()

        fire_gather(0, 0)
        fire_gather(1, 1)

        def step(t, r):
            @pl.when(t >= 2)
            def _():
                drain_write((r - 2) % NBUF)

            @pl.when(t + 2 < n_chunks)
            def _():
                fire_gather(t + 2, (r + 2) % NBUF)

            drain_gather(r)
            fire_write(t, r)

        def quad(i, carry):
            for r in range(NBUF):
                step(NBUF * i + r, r)
            return carry

        lax.fori_loop(0, n_chunks // NBUF, quad, 0)
        drain_write(2)
        drain_write(3)

    return k


def kernel(token_ids, weights):
    b, t = token_ids.shape
    dim = weights.shape[1]
    n_total = b * t
    n_workers = 32
    n_chunks = n_total // (n_workers * CHUNK)
    assert n_total % (n_workers * CHUNK) == 0 and n_chunks % NBUF == 0
    idx = token_ids.astype(jnp.int32).reshape(n_workers, n_chunks, CHUNK)
    wpad = jnp.pad(weights, ((0, 0), (0, PDIM - dim)))
    out = _build(n_total, n_workers, n_chunks)(idx, wpad)
    return out[:, :dim].reshape(b, t, dim)
